# speedup vs baseline: 2.2954x; 2.2954x over previous
import functools

import jax
import jax.numpy as jnp
from jax.experimental import pallas as pl

HALF = 64
ROTARY_DIM = 128
N_TOKENS = 32768
BLOCK = 1024


def _rope_body(p0_ref, p1_ref, p2_ref, inv_ref, cos_ref, sin_ref):
    p0 = p0_ref[:].astype(jnp.float32)[:, None]
    p1 = p1_ref[:].astype(jnp.float32)[:, None]
    p2 = p2_ref[:].astype(jnp.float32)[:, None]
    col = jax.lax.broadcasted_iota(jnp.int32, (1, HALF), 1)
    mod3 = col % 3
    psel = jnp.where(mod3 == 0, p0, jnp.where(mod3 == 1, p1, p2))
    angle = psel * inv_ref[:].reshape(1, HALF)
    c = jnp.cos(angle)
    s = jnp.sin(angle)
    cos_ref[:, :HALF] = c
    cos_ref[:, HALF:] = c
    sin_ref[:, :HALF] = s
    sin_ref[:, HALF:] = s


@jax.jit
def _rope(p0, p1, p2, inv_freq):
    grid = (N_TOKENS // BLOCK,)
    out_shape = [
        jax.ShapeDtypeStruct((N_TOKENS, ROTARY_DIM), jnp.float32),
        jax.ShapeDtypeStruct((N_TOKENS, ROTARY_DIM), jnp.float32),
    ]
    pos_spec = pl.BlockSpec((BLOCK,), lambda i: (i,))
    return pl.pallas_call(
        _rope_body,
        grid=grid,
        in_specs=[pos_spec, pos_spec, pos_spec,
                  pl.BlockSpec((HALF,), lambda i: (0,))],
        out_specs=[pl.BlockSpec((BLOCK, ROTARY_DIM), lambda i: (i, 0)),
                   pl.BlockSpec((BLOCK, ROTARY_DIM), lambda i: (i, 0))],
        out_shape=out_shape,
    )(p0, p1, p2, inv_freq)


def kernel(positions, inv_freq):
    cos, sin = _rope(positions[0], positions[1], positions[2], inv_freq)
    return cos, sin
